# baseline (speedup 1.0000x reference)
import functools

import jax
import jax.numpy as jnp
from jax import lax
from jax.experimental import pallas as pl
from jax.experimental.pallas import tpu as pltpu
from jax.experimental.pallas import tpu_sc as plsc

CROP_H = 7
CROP_W = 7
P = CROP_H * CROP_W

NC = 2
NS = 16
NW = NC * NS

LANES = 128
KROWS = 104
QG = 12


def _sc_crop(table, idx, w, n_boxes, c):
    bpw = -(-n_boxes // NW)
    n_lo = n_boxes // NW
    n_rem = n_boxes % NW
    cvecs = c // 16
    cp = c * P

    mesh = plsc.VectorSubcoreMesh(core_axis_name="c", subcore_axis_name="s")

    @functools.partial(
        pl.kernel,
        mesh=mesh,
        compiler_params=pltpu.CompilerParams(needs_layout_passes=False),
        out_type=jax.ShapeDtypeStruct((n_boxes * cp,), jnp.float32),
        scratch_types=[
            pltpu.VMEM((2 * bpw * LANES,), jnp.int32),
            pltpu.VMEM((2, 2 * LANES), jnp.float32),
            pltpu.VMEM((2, KROWS, LANES), jnp.float32),
            pltpu.VMEM((2, KROWS, LANES), jnp.float32),
            pltpu.VMEM((cp,), jnp.float32),
            pltpu.VMEM((cp,), jnp.float32),
            pltpu.SemaphoreType.DMA,
            pltpu.SemaphoreType.DMA,
            pltpu.SemaphoreType.DMA,
            pltpu.SemaphoreType.DMA,
            pltpu.SemaphoreType.DMA,
            pltpu.SemaphoreType.DMA,
        ],
    )
    def k(idx_hbm, w_hbm, table_hbm, out_hbm,
          idx_all, wbuf, ga, gb, acc0, acc1, sg0, sg1, sw0, sw1, so0, so1):
        wid = lax.axis_index("s") * NC + lax.axis_index("c")
        start = jnp.minimum(
            wid * n_lo + jnp.minimum(wid, n_rem), n_boxes - bpw
        )
        lane = lax.iota(jnp.int32, 16)
        lane49 = lane * P

        pltpu.sync_copy(
            idx_hbm.at[pl.ds(start * 2 * LANES, 2 * bpw * LANES)], idx_all
        )

        def issue(t, g, sg, wslot, sw):
            pltpu.async_copy(
                table_hbm.at[idx_all.at[pl.ds(t * 2 * LANES, KROWS)]], g.at[0], sg
            )
            pltpu.async_copy(
                table_hbm.at[idx_all.at[pl.ds(t * 2 * LANES + LANES, KROWS)]],
                g.at[1], sg,
            )
            pltpu.async_copy(
                w_hbm.at[pl.ds((start + t) * 2 * LANES, 2 * LANES)],
                wbuf.at[wslot], sw,
            )

        def wait_inputs(t, g, sg, wslot, sw):
            pltpu.make_async_copy(
                table_hbm.at[idx_all.at[pl.ds(t * 2 * LANES, KROWS)]], g.at[0], sg
            ).wait()
            pltpu.make_async_copy(
                table_hbm.at[idx_all.at[pl.ds(t * 2 * LANES + LANES, KROWS)]],
                g.at[1], sg,
            ).wait()
            pltpu.make_async_copy(
                w_hbm.at[pl.ds((start + t) * 2 * LANES, 2 * LANES)],
                wbuf.at[wslot], sw,
            ).wait()

        def interp_pos(p, w16, wq, g, acc):
            wtl = jnp.full((16,), w16[wq])
            wtr = jnp.full((16,), w16[wq + 1])
            wbl = jnp.full((16,), w16[wq + 2])
            wbr = jnp.full((16,), w16[wq + 3])
            flat = lane49 + p
            for cv in range(cvecs):
                sl = pl.ds(cv * 16, 16)
                val = (wtl * g[0, p, sl] + wtr * g[0, P + p, sl]
                       + wbl * g[1, p, sl] + wbr * g[1, P + p, sl])
                plsc.store_scatter(acc, [flat + cv * 16 * P], val)

        def compute(t, wslot, g, acc):
            @plsc.parallel_loop(0, QG, 1, unroll=2)
            def _(q):
                w16 = wbuf[wslot, pl.ds(q * 16, 16)]
                for kk in range(4):
                    interp_pos(q * 4 + kk, w16, 4 * kk, g, acc)

            w16 = wbuf[wslot, pl.ds(4 * QG * 4, 16)]
            interp_pos(4 * QG, w16, 0, g, acc)

        def box(t, g, sg, wslot, sw, acc, so, has_next, g_next, sg_next, sw_next):
            wait_inputs(t, g, sg, wslot, sw)
            if has_next:
                issue(t + 1, g_next, sg_next, 1 - wslot, sw_next)

            @pl.when(t >= 2)
            def _():
                pltpu.make_async_copy(
                    acc, out_hbm.at[pl.ds((start + t) * cp, cp)], so
                ).wait()

            compute(t, wslot, g, acc)
            pltpu.async_copy(acc, out_hbm.at[pl.ds((start + t) * cp, cp)], so)

        issue(0, ga, sg0, 0, sw0)

        def pair(u, _):
            t = 2 * u
            box(t, ga, sg0, 0, sw0, acc0, so0, True, gb, sg1, sw1)
            box(t + 1, gb, sg1, 1, sw1, acc1, so1, True, ga, sg0, sw0)
            return 0

        lax.fori_loop(0, (bpw - 1) // 2, pair, 0)
        box(bpw - 1, ga, sg0, 0, sw0, acc0, so0, False, None, None, None)
        pltpu.make_async_copy(acc0, out_hbm.at[pl.ds(start * cp, cp)], so0).wait()
        pltpu.make_async_copy(acc1, out_hbm.at[pl.ds(start * cp, cp)], so1).wait()

    return k(idx, w, table)


def kernel(image, boxes, box_ind):
    b, c, h, w = image.shape
    n = boxes.shape[0]

    table = jnp.pad(
        image.transpose(0, 2, 3, 1), ((0, 0), (0, 0), (0, 0), (0, LANES - c))
    ).reshape(b * h * w, LANES)

    y1 = boxes[:, 0]
    x1 = boxes[:, 1]
    y2 = boxes[:, 2]
    x2 = boxes[:, 3]
    ii = jnp.arange(CROP_H, dtype=jnp.float32)
    jj = jnp.arange(CROP_W, dtype=jnp.float32)
    h_scale = (y2 - y1) * (h - 1) / (CROP_H - 1)
    w_scale = (x2 - x1) * (w - 1) / (CROP_W - 1)
    in_y = y1[:, None] * (h - 1) + ii[None, :] * h_scale[:, None]
    in_x = x1[:, None] * (w - 1) + jj[None, :] * w_scale[:, None]
    vy = (in_y >= 0.0) & (in_y <= h - 1.0)
    vx = (in_x >= 0.0) & (in_x <= w - 1.0)
    in_y_c = jnp.clip(in_y, 0.0, h - 1.0)
    in_x_c = jnp.clip(in_x, 0.0, w - 1.0)
    ty = jnp.minimum(jnp.floor(in_y_c).astype(jnp.int32), h - 2)
    yl = in_y_c - ty.astype(jnp.float32)
    tx = jnp.minimum(jnp.floor(in_x_c).astype(jnp.int32), w - 2)
    xl = in_x_c - tx.astype(jnp.float32)

    base = (box_ind.astype(jnp.int32) * h)[:, None, None]
    r_tl = ((base + ty[:, :, None]) * w + tx[:, None, :]).reshape(n, P)
    tail = jnp.zeros((n, LANES - KROWS), jnp.int32)
    r_bl = r_tl + w
    s0 = jnp.concatenate([r_tl, r_tl + 1, r_tl[:, : KROWS - 2 * P], tail], axis=1)
    s1 = jnp.concatenate([r_bl, r_bl + 1, r_bl[:, : KROWS - 2 * P], tail], axis=1)
    idx = jnp.stack([s0, s1], axis=1).reshape(2 * n * LANES)

    valid = (vy[:, :, None] & vx[:, None, :]).reshape(n, P).astype(jnp.float32)
    oyl = (1.0 - yl)[:, :, None]
    oxl = (1.0 - xl)[:, None, :]
    yl3 = yl[:, :, None]
    xl3 = xl[:, None, :]
    wts = jnp.stack(
        [
            (oyl * oxl).reshape(n, P),
            (oyl * xl3).reshape(n, P),
            (yl3 * oxl).reshape(n, P),
            (yl3 * xl3).reshape(n, P),
        ],
        axis=2,
    ) * valid[:, :, None]
    wts = jnp.concatenate(
        [wts.reshape(n, 4 * P), jnp.zeros((n, 2 * LANES - 4 * P), jnp.float32)],
        axis=1,
    ).reshape(2 * n * LANES)

    out = _sc_crop(table, idx, wts, n, c)
    return out.reshape(n, c, CROP_H, CROP_W)
